# Initial kernel scaffold; baseline (speedup 1.0000x reference)
import jax
import jax.numpy as jnp
from jax.experimental import pallas as pl


def kernel(input_tensor, edge_index, W, att_src, att_dst, b):
    raise NotImplementedError("write your pallas kernel here")


# speedup vs baseline: 4.7593x; 4.7593x over previous
import functools

import jax
import jax.numpy as jnp
from jax import lax
from jax.experimental import pallas as pl
from jax.experimental.pallas import tpu as pltpu
from jax.experimental.pallas import tpu_sc as plsc

N_NODES = 10000
INPUT_DIM = 128
HIDDEN = 64
N_PAD = 10240
GCOLS = 128
BLK = 1024
N_TC_BLKS = N_PAD // BLK

NSC = 2
NTILES = 16
CHUNK = 16
ROWS_PER_TILE = N_PAD // NTILES
ZROWS = 64


def _round_up(x, m):
    return ((x + m - 1) // m) * m


def _proj_body(x_ref, h_ref, wx_ref, wh_ref, vx_ref, vh_ref, p_ref, av_ref):
    x = x_ref[...]
    h = h_ref[...]
    p_ref[...] = (jnp.dot(x, wx_ref[...], preferred_element_type=jnp.float32)
                  + jnp.dot(h, wh_ref[...], preferred_element_type=jnp.float32))
    av_ref[...] = (
        lax.dot_general(vx_ref[...], x, (((0,), (1,)), ((), ())),
                        preferred_element_type=jnp.float32)
        + lax.dot_general(vh_ref[...], h, (((0,), (1,)), ((), ())),
                          preferred_element_type=jnp.float32))


def _make_proj():
    return pl.pallas_call(
        _proj_body,
        grid=(N_TC_BLKS,),
        in_specs=[
            pl.BlockSpec((BLK, INPUT_DIM), lambda i: (i, 0)),
            pl.BlockSpec((BLK, HIDDEN), lambda i: (i, 0)),
            pl.BlockSpec((INPUT_DIM, GCOLS), lambda i: (0, 0)),
            pl.BlockSpec((HIDDEN, GCOLS), lambda i: (0, 0)),
            pl.BlockSpec((INPUT_DIM, 2), lambda i: (0, 0)),
            pl.BlockSpec((HIDDEN, 2), lambda i: (0, 0)),
        ],
        out_specs=[
            pl.BlockSpec((BLK, GCOLS), lambda i: (i, 0)),
            pl.BlockSpec((2, BLK), lambda i: (0, i)),
        ],
        out_shape=[
            jax.ShapeDtypeStruct((N_PAD, GCOLS), jnp.float32),
            jax.ShapeDtypeStruct((2, N_PAD), jnp.float32),
        ],
    )


def _make_edge_kernel(chunks_per_tile):
    ept = chunks_per_tile * CHUNK

    mesh = plsc.VectorSubcoreMesh(core_axis_name="c", subcore_axis_name="s",
                                  num_cores=NSC, num_subcores=NTILES)

    @functools.partial(
        pl.kernel,
        out_type=[
            jax.ShapeDtypeStruct((NSC, N_PAD, GCOLS), jnp.float32),
            jax.ShapeDtypeStruct((NSC, N_PAD), jnp.float32),
        ],
        mesh=mesh,
        compiler_params=pltpu.CompilerParams(needs_layout_passes=False),
        scratch_types=[
            pltpu.VMEM((ept,), jnp.int32),
            pltpu.VMEM((ept,), jnp.int32),
            pltpu.VMEM((N_PAD,), jnp.float32),
            pltpu.VMEM((N_PAD,), jnp.float32),
            pltpu.VMEM((CHUNK, GCOLS), jnp.float32),
            pltpu.VMEM((CHUNK,), jnp.float32),
            pltpu.VMEM((ZROWS, GCOLS), jnp.float32),
            pltpu.VMEM((ROWS_PER_TILE,), jnp.float32),
            pltpu.VMEM_SHARED((N_PAD, GCOLS), jnp.float32),
            pltpu.VMEM_SHARED((N_PAD,), jnp.float32),
        ],
    )
    def edge_kernel(p_hbm, av_hbm, src_hbm, dst_hbm, zrow_hbm, zden_hbm,
                    out_hbm, den_hbm,
                    src_v, dst_v, asrc_v, adst_v, rows_v, ex_v,
                    zrow_v, zden_v, out_s, den_s):
        c = lax.axis_index("c")
        s = lax.axis_index("s")

        pltpu.sync_copy(src_hbm.at[c, s], src_v)
        pltpu.sync_copy(dst_hbm.at[c, s], dst_v)
        pltpu.sync_copy(av_hbm.at[0], asrc_v)
        pltpu.sync_copy(av_hbm.at[1], adst_v)
        pltpu.sync_copy(zrow_hbm, zrow_v)
        pltpu.sync_copy(zden_hbm, zden_v)

        row0 = s * ROWS_PER_TILE
        for k in range(ROWS_PER_TILE // ZROWS):
            pltpu.sync_copy(zrow_v, out_s.at[pl.ds(row0 + k * ZROWS, ZROWS)])
        pltpu.sync_copy(zden_v, den_s.at[pl.ds(row0, ROWS_PER_TILE)])
        plsc.subcore_barrier()

        rowidx = lax.iota(jnp.int32, CHUNK)

        def chunk_body(ci, _):
            base = ci * CHUNK
            s_idx = src_v[pl.ds(base, CHUNK)]
            d_idx = dst_v[pl.ds(base, CHUNK)]
            a = (plsc.load_gather(asrc_v, [s_idx])
                 + plsc.load_gather(adst_v, [d_idx]))
            a = jnp.where(a >= 0.0, a, a * jnp.float32(0.2))
            ex = jnp.exp(a)
            ex_v[pl.ds(0, CHUNK)] = ex
            pltpu.sync_copy(p_hbm.at[s_idx], rows_v)
            for j in range(GCOLS):
                colj = jnp.full((CHUNK,), j, jnp.int32)
                col = plsc.load_gather(rows_v, [rowidx, colj])
                plsc.store_scatter(rows_v, [rowidx, colj], col * ex)
            pltpu.sync_copy(rows_v, out_s.at[d_idx], add=True)
            pltpu.sync_copy(ex_v, den_s.at[d_idx], add=True)
            return 0

        lax.fori_loop(0, chunks_per_tile, chunk_body, 0)
        plsc.subcore_barrier()

        for k in range(ROWS_PER_TILE // ZROWS):
            r = row0 + k * ZROWS
            pltpu.sync_copy(out_s.at[pl.ds(r, ZROWS)],
                            out_hbm.at[c, pl.ds(r, ZROWS)])
        pltpu.sync_copy(den_s.at[pl.ds(row0, ROWS_PER_TILE)],
                        den_hbm.at[c, pl.ds(row0, ROWS_PER_TILE)])

    return edge_kernel


def _gru_body(oa_ref, ob_ref, da_ref, db_ref, h_ref, b_ref, hn_ref):
    o = oa_ref[0] + ob_ref[0]
    den = da_ref[...] + db_ref[...] + jnp.float32(1e-16)
    o = o / den + b_ref[...]
    z = jax.nn.sigmoid(o[:, :HIDDEN])
    n = jnp.tanh(o[:, HIDDEN:])
    hn_ref[...] = (1.0 - z) * n + z * h_ref[...]


def _make_gru():
    return pl.pallas_call(
        _gru_body,
        grid=(N_TC_BLKS,),
        in_specs=[
            pl.BlockSpec((1, BLK, GCOLS), lambda i: (0, i, 0)),
            pl.BlockSpec((1, BLK, GCOLS), lambda i: (1, i, 0)),
            pl.BlockSpec((BLK, 1), lambda i: (i, 0)),
            pl.BlockSpec((BLK, 1), lambda i: (i, 0)),
            pl.BlockSpec((BLK, HIDDEN), lambda i: (i, 0)),
            pl.BlockSpec((1, GCOLS), lambda i: (0, 0)),
        ],
        out_specs=pl.BlockSpec((BLK, HIDDEN), lambda i: (i, 0)),
        out_shape=jax.ShapeDtypeStruct((N_PAD, HIDDEN), jnp.float32),
    )


def kernel(input_tensor, edge_index, W, att_src, att_dst, b):
    T, B, N, D = input_tensor.shape
    f32 = jnp.float32

    att2 = jnp.stack([att_src, att_dst], axis=1)
    V = W @ att2
    Vx, Vh = V[:INPUT_DIM], V[INPUT_DIM:]
    Wzn = W[:, HIDDEN:]
    Wx, Wh = Wzn[:INPUT_DIM], Wzn[INPUT_DIM:]
    b2 = b[HIDDEN:][None, :]

    loops = jnp.arange(N, dtype=jnp.int32)
    src = jnp.concatenate([edge_index[0].astype(jnp.int32), loops])
    dst = jnp.concatenate([edge_index[1].astype(jnp.int32), loops])
    n_edges = src.shape[0]
    per_tile = _round_up(n_edges, NSC * NTILES * CHUNK) // (NSC * NTILES)
    chunks_per_tile = per_tile // CHUNK
    e_pad = per_tile * NSC * NTILES
    pad_idx = N + (jnp.arange(e_pad - n_edges, dtype=jnp.int32) % (N_PAD - N))
    src_p = jnp.concatenate([src, pad_idx]).reshape(NSC, NTILES, per_tile)
    dst_p = jnp.concatenate([dst, pad_idx]).reshape(NSC, NTILES, per_tile)

    zrow = jnp.zeros((ZROWS, GCOLS), f32)
    zden = jnp.zeros((ROWS_PER_TILE,), f32)

    x = input_tensor[:, 0]
    x = jnp.pad(x, ((0, 0), (0, N_PAD - N), (0, 0)))
    h = jnp.zeros((N_PAD, HIDDEN), f32)

    proj = _make_proj()
    edge = _make_edge_kernel(chunks_per_tile)
    gru = _make_gru()

    outs = []
    for t in range(T):
        P, av = proj(x[t], h, Wx, Wh, Vx, Vh)
        out2, den2 = edge(P, av, src_p, dst_p, zrow, zden)
        h = gru(out2, out2, den2[0][:, None], den2[1][:, None], h, b2)
        outs.append(h[:N])

    layer_output = jnp.stack(outs, axis=0)[:, None]
    return (layer_output, h[:N][None])
